# baseline (speedup 1.0000x reference)
import functools

import jax
import jax.numpy as jnp
from jax import lax
from jax.experimental import pallas as pl
from jax.experimental.pallas import tpu as pltpu
from jax.experimental.pallas import tpu_sc as plsc


def _copy_kernel(te_ref, out_ref):
    out_ref[...] = te_ref[...]


def _make_sc_pos(batch, seq, d, dtype):
    info = plsc.get_sparse_core_info()
    nc, ns = info.num_cores, info.num_subcores
    nw = nc * ns
    per_b = nw // batch
    rows = seq // per_b
    chunk = 32
    n_it = rows // chunk
    mesh = plsc.VectorSubcoreMesh(core_axis_name="c", subcore_axis_name="s")

    @functools.partial(
        pl.kernel,
        mesh=mesh,
        out_type=jax.ShapeDtypeStruct((batch * seq, d), dtype),
        scratch_types=[
            pltpu.VMEM((2, chunk, d), dtype),
            pltpu.SemaphoreType.DMA,
            pltpu.SemaphoreType.DMA,
        ],
    )
    def sc_pos(w_hbm, out_hbm, vbuf, rsem, wsem):
        wid = lax.axis_index("s") * nc + lax.axis_index("c")
        b = wid // per_b
        s_base = (wid % per_b) * rows
        o_base = b * seq + s_base

        def read(i, slot):
            return pltpu.make_async_copy(
                w_hbm.at[pl.ds(s_base + i * chunk, chunk)], vbuf.at[slot], rsem)

        def write(i, slot):
            return pltpu.make_async_copy(
                vbuf.at[slot], out_hbm.at[pl.ds(o_base + i * chunk, chunk)], wsem)

        read(0, 0).start()
        for i in range(n_it):
            slot = i % 2
            read(i, slot).wait()
            if i + 1 < n_it:
                if i >= 1:
                    write(i - 1, (i - 1) % 2).wait()
                read(i + 1, (i + 1) % 2).start()
            write(i, slot).start()
        write(n_it - 2, (n_it - 2) % 2).wait()
        write(n_it - 1, (n_it - 1) % 2).wait()

    return sc_pos


def _dma_kernel(w_ref, te_ref, pos_ref, te_out_ref, sem):
    batch = pos_ref.shape[0]
    copies = [pltpu.make_async_copy(te_ref, te_out_ref, sem)]
    for b in range(batch):
        copies.append(pltpu.make_async_copy(w_ref, pos_ref.at[b], sem))
    for c in copies:
        c.start()
    for c in copies:
        c.wait()


def kernel(tokens, token_embed, W_pos):
    batch, seq, d = token_embed.shape
    w_sliced = W_pos[:seq]
    pos_embed, te_out = pl.pallas_call(
        _dma_kernel,
        in_specs=[
            pl.BlockSpec(memory_space=pl.ANY),
            pl.BlockSpec(memory_space=pl.ANY),
        ],
        out_specs=[
            pl.BlockSpec(memory_space=pl.ANY),
            pl.BlockSpec(memory_space=pl.ANY),
        ],
        out_shape=[
            jax.ShapeDtypeStruct((batch, seq, d), W_pos.dtype),
            jax.ShapeDtypeStruct((batch, seq, d), token_embed.dtype),
        ],
        scratch_shapes=[pltpu.SemaphoreType.DMA],
    )(w_sliced, token_embed)
    return (pos_embed, te_out)


# speedup vs baseline: 60.8340x; 60.8340x over previous
import functools

import jax
import jax.numpy as jnp
from jax import lax
from jax.experimental import pallas as pl
from jax.experimental.pallas import tpu as pltpu
from jax.experimental.pallas import tpu_sc as plsc


def _copy_kernel(te_ref, out_ref):
    out_ref[...] = te_ref[...]


def _make_sc_pos(batch, seq, d, dtype):
    info = plsc.get_sparse_core_info()
    nc, ns = info.num_cores, info.num_subcores
    nw = nc * ns
    per_b = nw // batch
    rows = seq // per_b
    chunk = 32
    n_it = rows // chunk
    mesh = plsc.VectorSubcoreMesh(core_axis_name="c", subcore_axis_name="s")

    @functools.partial(
        pl.kernel,
        mesh=mesh,
        out_type=jax.ShapeDtypeStruct((batch * seq, d), dtype),
        scratch_types=[
            pltpu.VMEM((2, chunk, d), dtype),
            pltpu.SemaphoreType.DMA,
            pltpu.SemaphoreType.DMA,
        ],
    )
    def sc_pos(w_hbm, out_hbm, vbuf, rsem, wsem):
        wid = lax.axis_index("s") * nc + lax.axis_index("c")
        b = wid // per_b
        s_base = (wid % per_b) * rows
        o_base = b * seq + s_base

        def read(i, slot):
            return pltpu.make_async_copy(
                w_hbm.at[pl.ds(s_base + i * chunk, chunk)], vbuf.at[slot], rsem)

        def write(i, slot):
            return pltpu.make_async_copy(
                vbuf.at[slot], out_hbm.at[pl.ds(o_base + i * chunk, chunk)], wsem)

        read(0, 0).start()
        for i in range(n_it):
            slot = i % 2
            read(i, slot).wait()
            if i + 1 < n_it:
                if i >= 1:
                    write(i - 1, (i - 1) % 2).wait()
                read(i + 1, (i + 1) % 2).start()
            write(i, slot).start()
        write(n_it - 2, (n_it - 2) % 2).wait()
        write(n_it - 1, (n_it - 1) % 2).wait()

    return sc_pos


def _dma_kernel(w_ref, te_ref, pos_ref, te_out_ref, sem):
    batch = pos_ref.shape[0]
    copies = [pltpu.make_async_copy(te_ref, te_out_ref, sem)]
    for b in range(batch):
        copies.append(pltpu.make_async_copy(w_ref, pos_ref.at[b], sem))
    for c in copies:
        c.start()
    for c in copies:
        c.wait()


def _fused_kernel(w_ref, te_ref, pos_ref, te_out_ref):
    pos_ref[...] = jnp.broadcast_to(w_ref[...][None, :, :], pos_ref.shape)
    te_out_ref[...] = te_ref[...]


def kernel(tokens, token_embed, W_pos):
    batch, seq, d = token_embed.shape
    block_s = 512
    pos_embed, te_out = pl.pallas_call(
        _fused_kernel,
        grid=(seq // block_s,),
        in_specs=[
            pl.BlockSpec((block_s, d), lambda j: (j, 0)),
            pl.BlockSpec((batch, block_s, d), lambda j: (0, j, 0)),
        ],
        out_specs=[
            pl.BlockSpec((batch, block_s, d), lambda j: (0, j, 0)),
            pl.BlockSpec((batch, block_s, d), lambda j: (0, j, 0)),
        ],
        out_shape=[
            jax.ShapeDtypeStruct((batch, seq, d), W_pos.dtype),
            jax.ShapeDtypeStruct((batch, seq, d), token_embed.dtype),
        ],
    )(W_pos, token_embed)
    return (pos_embed, te_out)
